# baseline (speedup 1.0000x reference)
import functools

import jax
import jax.numpy as jnp
import numpy as np
from jax import lax
from jax.experimental import pallas as pl
from jax.experimental.pallas import tpu as pltpu
from jax.experimental.pallas import tpu_sc as plsc

_N = 100000
_D = 128
_NUM_REMASK = 50000


def _rotl(x, d):
    return ((x << np.uint32(d)) | (x >> np.uint32(32 - d))).astype(np.uint32)


def _threefry2x32(k1, k2, x0, x1):
    rots = [[13, 15, 26, 6], [17, 29, 16, 24]]
    ks0, ks1 = np.uint32(k1), np.uint32(k2)
    ks2 = np.uint32(ks0 ^ ks1 ^ np.uint32(0x1BD11BDA))
    x0 = (x0 + ks0).astype(np.uint32)
    x1 = (x1 + ks1).astype(np.uint32)
    pairs = [(ks1, ks2), (ks2, ks0), (ks0, ks1), (ks1, ks2), (ks2, ks0)]
    for g in range(5):
        for d in rots[g % 2]:
            x0 = (x0 + x1).astype(np.uint32)
            x1 = _rotl(x1, d)
            x1 = (x1 ^ x0).astype(np.uint32)
        a, b = pairs[g]
        x0 = (x0 + a).astype(np.uint32)
        x1 = (x1 + b + np.uint32(g + 1)).astype(np.uint32)
    return x0, x1


def _np_permutation(seed, n):
    x = np.arange(n, dtype=np.int32)
    num_rounds = int(np.ceil(3 * np.log(max(1, n)) / np.log(2**32 - 1)))
    key = (np.uint32((seed >> 32) & 0xFFFFFFFF), np.uint32(seed & 0xFFFFFFFF))
    for _ in range(num_rounds):
        b1, b2 = _threefry2x32(*key, np.zeros(2, np.uint32), np.arange(2, dtype=np.uint32))
        key, subkey = (b1[0], b2[0]), (b1[1], b2[1])
        r1, r2 = _threefry2x32(*subkey, np.zeros(n, np.uint32), np.arange(n, dtype=np.uint32))
        x = x[np.argsort(r1 ^ r2, kind="stable")]
    return x


_PERM = _np_permutation(42, _N)
_REMASK_NP = _PERM[:_NUM_REMASK]
_REKEEP_NP = _PERM[_NUM_REMASK:]

_NC, _NS = 2, 16
_NW = _NC * _NS
_SLAB = 3128
_SLAB_LAST = _N - (_NW - 1) * _SLAB
_C = 112

_owner = np.minimum(_REMASK_NP // _SLAB, _NW - 1)
_buckets = [np.sort(_REMASK_NP[_owner == w]) for w in range(_NW)]
_PCH = max((len(b) + _C - 1) // _C for b in _buckets)
_PW = _PCH * _C


def _pad_bucket(b: np.ndarray) -> np.ndarray:
    return np.pad(b, (0, _PW - len(b)), mode="edge").astype(np.int32)


_RM_IDX = np.ascontiguousarray(
    np.stack([_pad_bucket(b) for b in _buckets]).reshape(_NW, _PCH, _C)
)

_MESH = plsc.VectorSubcoreMesh(
    core_axis_name="c", subcore_axis_name="s", num_cores=_NC, num_subcores=_NS
)


@functools.partial(
    pl.kernel,
    mesh=_MESH,
    out_type=jax.ShapeDtypeStruct((_N, _D), jnp.float32),
    scratch_types=[
        pltpu.VMEM((_PCH, _C), jnp.int32),
        pltpu.VMEM((_C,), jnp.int32),
        pltpu.VMEM((_C, _D), jnp.float32),
        pltpu.SemaphoreType.DMA,
        pltpu.SemaphoreType.DMA,
    ],
)
def _remask_sc(rep_hbm, tok_hbm, rm_hbm, out_hbm,
               rm_idx, zidx, tok_rows, sem_c, sem_t):
    wid = lax.axis_index("s") * _NC + lax.axis_index("c")
    base = pl.multiple_of(wid * _SLAB, 8)

    @pl.when(wid < _NW - 1)
    def _():
        pltpu.async_copy(
            rep_hbm.at[pl.ds(base, _SLAB)], out_hbm.at[pl.ds(base, _SLAB)],
            sem_c,
        )

    @pl.when(wid == _NW - 1)
    def _():
        pltpu.async_copy(
            rep_hbm.at[pl.ds(base, _SLAB_LAST)],
            out_hbm.at[pl.ds(base, _SLAB_LAST)],
            sem_c,
        )

    pltpu.sync_copy(rm_hbm.at[wid], rm_idx)
    zero = jnp.zeros((16,), jnp.int32)
    for j in range(_C // 16):
        zidx[pl.ds(j * 16, 16)] = zero
    pltpu.async_copy(tok_hbm.at[zidx], tok_rows, sem_t).wait()

    @pl.when(wid < _NW - 1)
    def _():
        pltpu.make_async_copy(
            rep_hbm.at[pl.ds(base, _SLAB)], out_hbm.at[pl.ds(base, _SLAB)],
            sem_c,
        ).wait()

    @pl.when(wid == _NW - 1)
    def _():
        pltpu.make_async_copy(
            rep_hbm.at[pl.ds(base, _SLAB_LAST)],
            out_hbm.at[pl.ds(base, _SLAB_LAST)],
            sem_c,
        ).wait()
    rm_copies = [
        pltpu.async_copy(tok_rows, out_hbm.at[rm_idx.at[j]], sem_t)
        for j in range(_PCH)
    ]
    for c in rm_copies:
        c.wait()


def kernel(rep, dec_mask_token):
    out = _remask_sc(rep, dec_mask_token, jnp.asarray(_RM_IDX))
    return (out, jnp.asarray(_REMASK_NP), jnp.asarray(_REKEEP_NP))
